# baseline (speedup 1.0000x reference)
import jax
import jax.numpy as jnp
from jax.experimental import pallas as pl
from jax.experimental.pallas import tpu as pltpu

_SCHED = (
    (3,   64,  64, 64, False),
    (64,  64,  64, 64, True),
    (64,  128, 32, 32, False),
    (128, 128, 32, 32, True),
    (128, 256, 16, 16, False),
    (256, 256, 16, 16, False),
    (256, 256, 16, 16, True),
)
_CF = 256


def _scatter9(dst, y, cout):
    h, w = y.shape[0], y.shape[1]
    z_row = jnp.zeros((1, w, 9 * cout), dst.dtype)
    z_col = jnp.zeros((h, 1, 9 * cout), dst.dtype)
    dst[0:1, :, :] = z_row
    dst[h - 1:h, :, :] = z_row
    dst[:, 0:1, :] = z_col
    dst[:, w - 1:w, :] = z_col
    for dy in range(3):
        r0, r1 = max(0, 1 - dy), min(h, h + 1 - dy)
        s0 = max(0, dy - 1)
        for dx in range(3):
            t = dy * 3 + dx
            c0, c1 = max(0, 1 - dx), min(w, w + 1 - dx)
            u0 = max(0, dx - 1)
            dst[r0:r1, c0:c1, t * cout:(t + 1) * cout] = (
                y[s0:s0 + (r1 - r0), u0:u0 + (c1 - c0), :])


def _pool2x2(y):
    h, w, c = y.shape
    y = jnp.max(y.reshape(h // 2, 2, w, c), axis=1)
    y = jnp.max(y.reshape(h // 2, w // 2, 2, c), axis=2)
    return y


def _backbone_kernel(x_ref, w0_ref, b0_ref, *args):
    wrefs = args[0:6]
    brefs = args[6:12]
    o_ref = args[12]
    cbufs = args[13:20]
    c1 = cbufs[0]

    x = x_ref[0, 0]
    for dx in range(3):
        c1[:, :, 3 * dx:3 * dx + 3] = x[:, dx:dx + 64, :]
    _, cout0, h0, w0, _ = _SCHED[0]
    f1 = c1[...].reshape((h0 + 2) * w0, 9)
    acc = None
    for dy in range(3):
        part = jnp.dot(f1[w0 * dy:w0 * dy + h0 * w0], w0_ref[0, dy],
                       preferred_element_type=jnp.float32)
        acc = part if acc is None else acc + part
    y = jnp.maximum(acc + b0_ref[0], 0.0).reshape(h0, w0, cout0)
    _scatter9(cbufs[1], y.astype(jnp.bfloat16), cout0)

    for li in range(1, 7):
        cin, cout, h, w, pool = _SCHED[li]
        acc = jnp.dot(cbufs[li][...].reshape(h * w, 9 * cin), wrefs[li - 1][0],
                      preferred_element_type=jnp.float32)
        y = jnp.maximum(acc + brefs[li - 1][0], 0.0).reshape(h, w, cout)
        if pool:
            y = _pool2x2(y)
        if li < 6:
            _scatter9(cbufs[li + 1], y.astype(jnp.bfloat16), cout)
        else:
            o_ref[...] = jnp.mean(y, axis=(0, 1)).reshape(1, 1, 1, _CF)


def _head_kernel(p_ref, w_ref, b_ref, o_ref):
    p = p_ref[...].reshape(p_ref.shape[0], p_ref.shape[1], _CF)
    m = jnp.max(p, axis=1)
    o_ref[...] = b_ref[...] + jnp.sum(m * w_ref[...]).reshape(1, 1)


def kernel(x0, x1, x2, w0, b0, w1, b1, w2, b2, w3, b3, w4, b4, w5, b5, w6, b6,
           fc_w, fc_b):
    xs = [jnp.squeeze(v, axis=0) for v in (x0, x1, x2)]
    x = jnp.stack(xs, axis=0).transpose(0, 1, 3, 4, 2)
    V, S, H, W, _ = x.shape
    xp = jnp.pad(x, ((0, 0), (0, 0), (1, 1), (1, 1), (0, 0))
                 ).astype(jnp.bfloat16)

    w0r = w0.reshape(V, 3, 9, _SCHED[0][1])
    wcat = []
    for wl, (cin, cout, _, _, _) in zip((w1, w2, w3, w4, w5, w6), _SCHED[1:]):
        wcat.append(wl.reshape(V, 9 * cin, cout))
    biases = (b1, b2, b3, b4, b5, b6)

    nprog = V * S
    iv = lambda p: (p // S, 0, 0)
    in_specs = [pl.BlockSpec((1, 1, H + 2, W + 2, 3),
                             lambda p: (p // S, p % S, 0, 0, 0)),
                pl.BlockSpec((1, 3, 9, _SCHED[0][1]), lambda p: (p // S, 0, 0, 0)),
                pl.BlockSpec((1, 1, _SCHED[0][1]), iv)]
    operands = [xp, w0r, b0]
    for wl, (cin, cout, _, _, _) in zip(wcat, _SCHED[1:]):
        in_specs.append(pl.BlockSpec((1, 9 * cin, cout), iv))
        operands.append(wl)
    for bl, (_, cout, _, _, _) in zip(biases, _SCHED[1:]):
        in_specs.append(pl.BlockSpec((1, 1, cout), iv))
        operands.append(bl)

    scratch = [pltpu.VMEM((H + 2, W, 9), jnp.bfloat16)]
    for cin, _, h, w, _ in _SCHED[1:]:
        scratch.append(pltpu.VMEM((h, w, 9 * cin), jnp.bfloat16))

    pooled = pl.pallas_call(
        _backbone_kernel,
        out_shape=jax.ShapeDtypeStruct((V, S, 1, _CF), jnp.float32),
        grid=(nprog,),
        in_specs=in_specs,
        out_specs=pl.BlockSpec((1, 1, 1, _CF), lambda p: (p // S, p % S, 0, 0)),
        scratch_shapes=scratch,
        compiler_params=pltpu.CompilerParams(
            dimension_semantics=("parallel",),
            vmem_limit_bytes=48 * 1024 * 1024),
    )(*operands)

    return pl.pallas_call(
        _head_kernel,
        out_shape=jax.ShapeDtypeStruct((1, 1), jnp.float32),
        in_specs=[pl.BlockSpec(memory_space=pltpu.MemorySpace.VMEM)] * 3,
        out_specs=pl.BlockSpec(memory_space=pltpu.MemorySpace.VMEM),
    )(pooled, fc_w, fc_b)
